# baseline (speedup 1.0000x reference)
import functools

import jax
import jax.numpy as jnp
from jax import lax
from jax.experimental import pallas as pl
from jax.experimental.pallas import tpu as pltpu
from jax.experimental.pallas import tpu_sc as plsc

N_SAMPLES = 3_200_000
N_RAYS = 100_000
NC = 2
NS = 16
NW = NC * NS
C_PER_W = N_SAMPLES // NW
CHUNK = 4_000
N_CHUNKS = C_PER_W // CHUNK
ACC_ROWS = 100_352
STRIPE = ACC_ROWS // NS
LANES = 16
DUMMY = ACC_ROWS - 1
IPAD = CHUNK + LANES
SCAP = 2 * CHUNK + 2 * LANES
BN = 5_120


def _tc_weighted_planes(normals, weights):
    def body(n_ref, w_ref, ox, oy, oz):
        p = n_ref[...] * w_ref[...]
        ox[...] = p[:, 0:1]
        oy[...] = p[:, 1:2]
        oz[...] = p[:, 2:3]

    return pl.pallas_call(
        body,
        grid=(N_SAMPLES // BN,),
        in_specs=[pl.BlockSpec((BN, 3), lambda i: (i, 0)),
                  pl.BlockSpec((BN, 1), lambda i: (i, 0))],
        out_specs=[pl.BlockSpec((BN, 1), lambda i: (i, 0))] * 3,
        out_shape=[jax.ShapeDtypeStruct((N_SAMPLES, 1), jnp.float32)] * 3,
    )(normals, weights)


def _sc_segment_sum(px, py, pz, idx_flat):
    @functools.partial(
        pl.kernel,
        out_type=jax.ShapeDtypeStruct((NC * 3, 1, ACC_ROWS), jnp.float32),
        mesh=plsc.VectorSubcoreMesh(core_axis_name="c", subcore_axis_name="s"),
        compiler_params=pltpu.CompilerParams(needs_layout_passes=False),
        scratch_types=[
            pltpu.MemorySpace.VMEM_SHARED((ACC_ROWS,), jnp.float32),
            pltpu.MemorySpace.VMEM_SHARED((ACC_ROWS,), jnp.float32),
            pltpu.MemorySpace.VMEM_SHARED((ACC_ROWS,), jnp.float32),
            pltpu.VMEM((STRIPE,), jnp.float32),
            pltpu.VMEM((2 * CHUNK,), jnp.float32),
            pltpu.VMEM((2 * CHUNK,), jnp.float32),
            pltpu.VMEM((2 * CHUNK,), jnp.float32),
            pltpu.VMEM((2 * IPAD,), jnp.int32),
            pltpu.VMEM((SCAP,), jnp.int32),
            pltpu.VMEM((SCAP,), jnp.float32),
            pltpu.VMEM((SCAP,), jnp.float32),
            pltpu.VMEM((SCAP,), jnp.float32),
            pltpu.SemaphoreType.DMA,
            pltpu.SemaphoreType.DMA,
        ],
    )
    def k(px_hbm, py_hbm, pz_hbm, i_hbm, out_hbm,
          acc_x, acc_y, acc_z, zbuf, vx, vy, vz, idx_f,
          st_id, st_x, st_y, st_z, sem_in, sem_s):
        cc = lax.axis_index("c")
        ss = lax.axis_index("s")
        wid = ss * NC + cc
        iota = lax.iota(jnp.int32, LANES)
        zeros = jnp.zeros((LANES,), jnp.float32)
        dummyv = jnp.full((LANES,), DUMMY, jnp.int32)
        accs = (acc_x, acc_y, acc_z)
        planes = (px_hbm, py_hbm, pz_hbm)
        vplanes = (vx, vy, vz)
        stages = (st_x, st_y, st_z)

        def in_copies(kk, b):
            base = wid * C_PER_W + kk * CHUNK
            cps = [
                pltpu.make_async_copy(
                    planes[c].at[pl.ds(base, CHUNK)],
                    vplanes[c].at[pl.ds(b * CHUNK, CHUNK)], sem_in)
                for c in range(3)]
            cps.append(pltpu.make_async_copy(
                i_hbm.at[pl.ds(base, CHUNK)],
                idx_f.at[pl.ds(b * IPAD, CHUNK)], sem_in))
            return cps

        def zbody(p, carry):
            zbuf[pl.ds(p * LANES, LANES)] = zeros
            return carry

        lax.fori_loop(0, STRIPE // LANES, zbody, 0)
        for a in accs:
            pltpu.sync_copy(zbuf, a.at[pl.ds(ss * STRIPE, STRIPE)])
        plsc.subcore_barrier()

        for cp in in_copies(0, 0):
            cp.start()

        def chunk_body(kk, carry):
            b = lax.rem(kk, 2)
            ib = b * IPAD
            for cp in in_copies(kk, b):
                cp.wait()

            @pl.when(kk + 1 < N_CHUNKS)
            def _():
                for cp in in_copies(kk + 1, 1 - b):
                    cp.start()

            plsc.store_scatter(idx_f, [ib + CHUNK + iota], dummyv)

            def jbody(j, st):
                ptr, cx, cy, cz = st
                off = j * LANES
                idv = idx_f[pl.ds(ib + off, LANES)]
                idn = plsc.load_gather(idx_f, [ib + off + 1 + iota])
                m = idv != idn
                mi = m.astype(jnp.int32)
                cnt = jnp.sum(mi)
                pos = ptr + plsc.cumsum(mi) - mi
                pos2 = pos + cnt
                plsc.store_scatter(st_id, [pos], idv, mask=m)
                plsc.store_scatter(st_id, [pos2], idn, mask=m)
                carries = []
                for c, (stg, cold) in enumerate(zip(stages, (cx, cy, cz))):
                    vc = vplanes[c][pl.ds(b * CHUNK + off, LANES)]
                    csb = plsc.cumsum(vc) + cold
                    plsc.store_scatter(stg, [pos], csb, mask=m)
                    plsc.store_scatter(stg, [pos2], -csb, mask=m)
                    carries.append(cold + jnp.sum(vc))
                return (ptr + 2 * cnt, carries[0], carries[1], carries[2])

            ptr, _, _, _ = lax.fori_loop(
                0, CHUNK // LANES, jbody,
                (jnp.int32(0), 0.0, 0.0, 0.0))

            plsc.store_scatter(st_id, [ptr + iota], dummyv)
            plsc.store_scatter(st_id, [ptr + LANES + iota], dummyv)

            ng = (ptr + LANES - 1) // LANES

            def gbody(g, c3):
                p = g * LANES
                idg = st_id[pl.ds(p, LANES)]
                for c in range(3):
                    pltpu.async_copy(stages[c].at[pl.ds(p, LANES)],
                                     accs[c].at[idg], sem_s, add=True)
                return c3

            lax.fori_loop(0, ng, gbody, 0)

            def dbody(g, c4):
                p = g * LANES
                idg = st_id[pl.ds(p, LANES)]
                for c in range(3):
                    pltpu.make_async_copy(stages[c].at[pl.ds(p, LANES)],
                                          accs[c].at[idg], sem_s).wait()
                return c4

            lax.fori_loop(0, ng, dbody, 0)
            return carry

        lax.fori_loop(0, N_CHUNKS, chunk_body, 0)

        plsc.subcore_barrier()
        for c in range(3):
            pltpu.sync_copy(
                accs[c].at[pl.ds(ss * STRIPE, STRIPE)],
                out_hbm.at[cc * 3 + c, 0, pl.ds(ss * STRIPE, STRIPE)])

    return k(px, py, pz, idx_flat)


def _merge_normalize(partial):
    BR = 512

    def body(x_ref, o_ref):
        x = x_ref[...]
        s = x[0] + x[1]
        nsq = jnp.sum(s * s, axis=0, keepdims=True)
        o_ref[...] = s / jnp.sqrt(jnp.maximum(nsq, 1e-20))

    return pl.pallas_call(
        body,
        grid=(ACC_ROWS // BR,),
        in_specs=[pl.BlockSpec((NC, 3, BR), lambda i: (0, 0, i))],
        out_specs=pl.BlockSpec((3, BR), lambda i: (0, i)),
        out_shape=jax.ShapeDtypeStruct((3, ACC_ROWS), jnp.float32),
    )(partial)


def kernel(normals, weights, ray_indices, num_rays):
    idx = ray_indices.astype(jnp.int32)
    px, py, pz = _tc_weighted_planes(normals, weights)
    partial = _sc_segment_sum(px.reshape(N_SAMPLES), py.reshape(N_SAMPLES),
                              pz.reshape(N_SAMPLES), idx)
    merged = _merge_normalize(partial.reshape(NC, 3, ACC_ROWS))
    return merged[:, :N_RAYS].T


# speedup vs baseline: 10.7724x; 10.7724x over previous
import functools

import jax
import jax.numpy as jnp
from jax import lax
from jax.experimental import pallas as pl
from jax.experimental.pallas import tpu as pltpu
from jax.experimental.pallas import tpu_sc as plsc

N_SAMPLES = 3_200_000
N_RAYS = 100_000
NC = 2
NS = 16
NW = NC * NS
C_PER_W = N_SAMPLES // NW
CHUNK = 4_000
N_CHUNKS = C_PER_W // CHUNK
ACC_ROWS = 100_352
STRIPE = ACC_ROWS // NS
LANES = 16
DUMMY = ACC_ROWS - 1
IPAD = CHUNK + LANES
SCAP = 2 * CHUNK + 2 * LANES


def _sc_segment_sum(nx, ny, nz, weights_flat, idx_flat):
    @functools.partial(
        pl.kernel,
        out_type=jax.ShapeDtypeStruct((NC * 3, 1, ACC_ROWS), jnp.float32),
        mesh=plsc.VectorSubcoreMesh(core_axis_name="c", subcore_axis_name="s"),
        compiler_params=pltpu.CompilerParams(needs_layout_passes=False),
        scratch_types=[
            pltpu.MemorySpace.VMEM_SHARED((ACC_ROWS,), jnp.float32),
            pltpu.MemorySpace.VMEM_SHARED((ACC_ROWS,), jnp.float32),
            pltpu.MemorySpace.VMEM_SHARED((ACC_ROWS,), jnp.float32),
            pltpu.VMEM((STRIPE,), jnp.float32),
            pltpu.VMEM((2 * CHUNK,), jnp.float32),
            pltpu.VMEM((2 * CHUNK,), jnp.float32),
            pltpu.VMEM((2 * CHUNK,), jnp.float32),
            pltpu.VMEM((2 * CHUNK,), jnp.float32),
            pltpu.VMEM((2 * IPAD,), jnp.int32),
            pltpu.VMEM((SCAP,), jnp.int32),
            pltpu.VMEM((SCAP,), jnp.float32),
            pltpu.VMEM((SCAP,), jnp.float32),
            pltpu.VMEM((SCAP,), jnp.float32),
            pltpu.SemaphoreType.DMA,
            pltpu.SemaphoreType.DMA,
        ],
    )
    def k(nx_hbm, ny_hbm, nz_hbm, w_hbm, i_hbm, out_hbm,
          acc_x, acc_y, acc_z, zbuf, vx, vy, vz, w_v, idx_f,
          st_id, st_x, st_y, st_z, sem_in, sem_s):
        cc = lax.axis_index("c")
        ss = lax.axis_index("s")
        wid = ss * NC + cc
        iota = lax.iota(jnp.int32, LANES)
        zeros = jnp.zeros((LANES,), jnp.float32)
        dummyv = jnp.full((LANES,), DUMMY, jnp.int32)
        accs = (acc_x, acc_y, acc_z)
        stages = (st_x, st_y, st_z)
        planes = (nx_hbm, ny_hbm, nz_hbm)
        vplanes = (vx, vy, vz)

        def in_copies(kk, b):
            base = wid * C_PER_W + kk * CHUNK
            cps = [
                pltpu.make_async_copy(
                    planes[c].at[pl.ds(base, CHUNK)],
                    vplanes[c].at[pl.ds(b * CHUNK, CHUNK)], sem_in)
                for c in range(3)]
            cps.append(pltpu.make_async_copy(
                w_hbm.at[pl.ds(base, CHUNK)],
                w_v.at[pl.ds(b * CHUNK, CHUNK)], sem_in))
            cps.append(pltpu.make_async_copy(
                i_hbm.at[pl.ds(base, CHUNK)],
                idx_f.at[pl.ds(b * IPAD, CHUNK)], sem_in))
            return cps

        def zbody(p, carry):
            zbuf[pl.ds(p * LANES, LANES)] = zeros
            return carry

        lax.fori_loop(0, STRIPE // LANES, zbody, 0)
        for a in accs:
            pltpu.sync_copy(zbuf, a.at[pl.ds(ss * STRIPE, STRIPE)])
        plsc.subcore_barrier()

        for cp in in_copies(0, 0):
            cp.start()

        def chunk_body(kk, carry):
            b = lax.rem(kk, 2)
            ib = b * IPAD
            for cp in in_copies(kk, b):
                cp.wait()

            @pl.when(kk + 1 < N_CHUNKS)
            def _():
                for cp in in_copies(kk + 1, 1 - b):
                    cp.start()

            plsc.store_scatter(idx_f, [ib + CHUNK + iota], dummyv)

            def jbody(j, st):
                ptr, cx, cy, cz = st
                off = j * LANES
                idv = idx_f[pl.ds(ib + off, LANES)]
                idn = plsc.load_gather(idx_f, [ib + off + 1 + iota])
                wv = w_v[pl.ds(b * CHUNK + off, LANES)]
                m = idv != idn
                mi = m.astype(jnp.int32)
                cnt = jnp.sum(mi)
                pos = ptr + plsc.cumsum(mi) - mi
                pos2 = pos + cnt
                plsc.store_scatter(st_id, [pos], idv, mask=m)
                plsc.store_scatter(st_id, [pos2], idn, mask=m)
                carries = []
                for c, (stg, cold) in enumerate(zip(stages, (cx, cy, cz))):
                    vc = wv * vplanes[c][pl.ds(b * CHUNK + off, LANES)]
                    csb = plsc.cumsum(vc) + cold
                    plsc.store_scatter(stg, [pos], csb, mask=m)
                    plsc.store_scatter(stg, [pos2], -csb, mask=m)
                    carries.append(cold + jnp.sum(vc))
                return (ptr + 2 * cnt, carries[0], carries[1], carries[2])

            ptr, _, _, _ = lax.fori_loop(
                0, CHUNK // LANES, jbody,
                (jnp.int32(0), 0.0, 0.0, 0.0))

            plsc.store_scatter(st_id, [ptr + iota], dummyv)
            plsc.store_scatter(st_id, [ptr + LANES + iota], dummyv)

            ng = (ptr + LANES - 1) // LANES

            def gbody(g, c3):
                p = g * LANES
                idg = st_id[pl.ds(p, LANES)]
                for c in range(3):
                    pltpu.async_copy(stages[c].at[pl.ds(p, LANES)],
                                     accs[c].at[idg], sem_s, add=True)
                return c3

            lax.fori_loop(0, ng, gbody, 0)

            def dbody(g, c4):
                p = g * LANES
                idg = st_id[pl.ds(p, LANES)]
                for c in range(3):
                    pltpu.make_async_copy(stages[c].at[pl.ds(p, LANES)],
                                          accs[c].at[idg], sem_s).wait()
                return c4

            lax.fori_loop(0, ng, dbody, 0)
            return carry

        lax.fori_loop(0, N_CHUNKS, chunk_body, 0)

        plsc.subcore_barrier()
        for c in range(3):
            pltpu.sync_copy(
                accs[c].at[pl.ds(ss * STRIPE, STRIPE)],
                out_hbm.at[cc * 3 + c, 0, pl.ds(ss * STRIPE, STRIPE)])

    return k(nx, ny, nz, weights_flat, idx_flat)


def _merge_normalize(partial):
    BR = 512

    def body(x_ref, o_ref):
        x = x_ref[...]
        s = x[0] + x[1]
        nsq = jnp.sum(s * s, axis=0, keepdims=True)
        o_ref[...] = s / jnp.sqrt(jnp.maximum(nsq, 1e-20))

    return pl.pallas_call(
        body,
        grid=(ACC_ROWS // BR,),
        in_specs=[pl.BlockSpec((NC, 3, BR), lambda i: (0, 0, i))],
        out_specs=pl.BlockSpec((3, BR), lambda i: (0, i)),
        out_shape=jax.ShapeDtypeStruct((3, ACC_ROWS), jnp.float32),
    )(partial)


def kernel(normals, weights, ray_indices, num_rays):
    idx = ray_indices.astype(jnp.int32)
    nx, ny, nz = (normals[:, 0], normals[:, 1], normals[:, 2])
    partial = _sc_segment_sum(nx, ny, nz, weights.reshape(N_SAMPLES), idx)
    merged = _merge_normalize(partial.reshape(NC, 3, ACC_ROWS))
    return merged[:, :N_RAYS].T


# speedup vs baseline: 17.1713x; 1.5940x over previous
import functools

import jax
import jax.numpy as jnp
from jax import lax
from jax.experimental import pallas as pl
from jax.experimental.pallas import tpu as pltpu
from jax.experimental.pallas import tpu_sc as plsc

N_SAMPLES = 3_200_000
N_RAYS = 100_000
NC = 2
NS = 16
NW = NC * NS
C_PER_W = N_SAMPLES // NW
CHUNK = 4_000
N_CHUNKS = C_PER_W // CHUNK
ACC_ROWS = 100_352
STRIPE = ACC_ROWS // NS
LANES = 16
DUMMY = ACC_ROWS - 1
IPAD = CHUNK + LANES
SCAP = 2 * CHUNK + 2 * LANES


def _sc_segment_sum(nx, ny, nz, weights_flat, idx_flat):
    @functools.partial(
        pl.kernel,
        out_type=jax.ShapeDtypeStruct((NC * 3, 1, ACC_ROWS), jnp.float32),
        mesh=plsc.VectorSubcoreMesh(core_axis_name="c", subcore_axis_name="s"),
        compiler_params=pltpu.CompilerParams(needs_layout_passes=False),
        scratch_types=[
            pltpu.MemorySpace.VMEM_SHARED((ACC_ROWS,), jnp.float32),
            pltpu.MemorySpace.VMEM_SHARED((ACC_ROWS,), jnp.float32),
            pltpu.MemorySpace.VMEM_SHARED((ACC_ROWS,), jnp.float32),
            pltpu.VMEM((STRIPE,), jnp.float32),
            pltpu.VMEM((2 * CHUNK,), jnp.float32),
            pltpu.VMEM((2 * CHUNK,), jnp.float32),
            pltpu.VMEM((2 * CHUNK,), jnp.float32),
            pltpu.VMEM((2 * CHUNK,), jnp.float32),
            pltpu.VMEM((2 * IPAD,), jnp.int32),
            pltpu.VMEM((SCAP,), jnp.int32),
            pltpu.VMEM((SCAP,), jnp.float32),
            pltpu.VMEM((SCAP,), jnp.float32),
            pltpu.VMEM((SCAP,), jnp.float32),
            pltpu.SemaphoreType.DMA,
            pltpu.SemaphoreType.DMA,
        ],
    )
    def k(nx_hbm, ny_hbm, nz_hbm, w_hbm, i_hbm, out_hbm,
          acc_x, acc_y, acc_z, zbuf, vx, vy, vz, w_v, idx_f,
          st_id, st_x, st_y, st_z, sem_in, sem_s):
        cc = lax.axis_index("c")
        ss = lax.axis_index("s")
        wid = ss * NC + cc
        iota = lax.iota(jnp.int32, LANES)
        zeros = jnp.zeros((LANES,), jnp.float32)
        dummyv = jnp.full((LANES,), DUMMY, jnp.int32)
        accs = (acc_x, acc_y, acc_z)
        stages = (st_x, st_y, st_z)
        planes = (nx_hbm, ny_hbm, nz_hbm)
        vplanes = (vx, vy, vz)

        def in_copies(kk, b):
            base = wid * C_PER_W + kk * CHUNK
            cps = [
                pltpu.make_async_copy(
                    planes[c].at[pl.ds(base, CHUNK)],
                    vplanes[c].at[pl.ds(b * CHUNK, CHUNK)], sem_in)
                for c in range(3)]
            cps.append(pltpu.make_async_copy(
                w_hbm.at[pl.ds(base, CHUNK)],
                w_v.at[pl.ds(b * CHUNK, CHUNK)], sem_in))
            cps.append(pltpu.make_async_copy(
                i_hbm.at[pl.ds(base, CHUNK)],
                idx_f.at[pl.ds(b * IPAD, CHUNK)], sem_in))
            return cps

        def zbody(p, carry):
            zbuf[pl.ds(p * LANES, LANES)] = zeros
            return carry

        lax.fori_loop(0, STRIPE // LANES, zbody, 0)
        for a in accs:
            pltpu.sync_copy(zbuf, a.at[pl.ds(ss * STRIPE, STRIPE)])
        plsc.subcore_barrier()

        for cp in in_copies(0, 0):
            cp.start()

        def chunk_body(kk, carry):
            b = lax.rem(kk, 2)
            ib = b * IPAD
            for cp in in_copies(kk, b):
                cp.wait()

            @pl.when(kk + 1 < N_CHUNKS)
            def _():
                for cp in in_copies(kk + 1, 1 - b):
                    cp.start()

            plsc.store_scatter(idx_f, [ib + CHUNK + iota], dummyv)

            def jbody(j, st):
                ptr, cx, cy, cz = st
                off = j * LANES
                idv = idx_f[pl.ds(ib + off, LANES)]
                idn = plsc.load_gather(idx_f, [ib + off + 1 + iota])
                wv = w_v[pl.ds(b * CHUNK + off, LANES)]
                m = idv != idn
                mi = m.astype(jnp.int32)
                cnt = jnp.sum(mi)
                pos = ptr + plsc.cumsum(mi) - mi
                pos2 = pos + cnt
                plsc.store_scatter(st_id, [pos], idv, mask=m)
                plsc.store_scatter(st_id, [pos2], idn, mask=m)
                carries = []
                for c, (stg, cold) in enumerate(zip(stages, (cx, cy, cz))):
                    vc = wv * vplanes[c][pl.ds(b * CHUNK + off, LANES)]
                    csb = plsc.cumsum(vc) + cold
                    plsc.store_scatter(stg, [pos], csb, mask=m)
                    plsc.store_scatter(stg, [pos2], -csb, mask=m)
                    carries.append(cold + jnp.sum(vc))
                return (ptr + 2 * cnt, carries[0], carries[1], carries[2])

            ptr, _, _, _ = plsc.parallel_loop(
                0, CHUNK // LANES, 1, unroll=4,
                carry=(jnp.int32(0), jnp.float32(0.0), jnp.float32(0.0),
                       jnp.float32(0.0)))(jbody)

            plsc.store_scatter(st_id, [ptr + iota], dummyv)
            plsc.store_scatter(st_id, [ptr + LANES + iota], dummyv)

            ng = (ptr + LANES - 1) // LANES

            def gbody(g, c3):
                p = g * LANES
                idg = st_id[pl.ds(p, LANES)]
                for c in range(3):
                    pltpu.async_copy(stages[c].at[pl.ds(p, LANES)],
                                     accs[c].at[idg], sem_s, add=True)
                return c3

            lax.fori_loop(0, ng, gbody, 0)

            def dbody(g, c4):
                p = g * LANES
                idg = st_id[pl.ds(p, LANES)]
                for c in range(3):
                    pltpu.make_async_copy(stages[c].at[pl.ds(p, LANES)],
                                          accs[c].at[idg], sem_s).wait()
                return c4

            lax.fori_loop(0, ng, dbody, 0)
            return carry

        lax.fori_loop(0, N_CHUNKS, chunk_body, 0)

        plsc.subcore_barrier()
        for c in range(3):
            pltpu.sync_copy(
                accs[c].at[pl.ds(ss * STRIPE, STRIPE)],
                out_hbm.at[cc * 3 + c, 0, pl.ds(ss * STRIPE, STRIPE)])

    return k(nx, ny, nz, weights_flat, idx_flat)


def _merge_normalize(partial):
    BR = 512

    def body(x_ref, o_ref):
        x = x_ref[...]
        s = x[0] + x[1]
        nsq = jnp.sum(s * s, axis=0, keepdims=True)
        o_ref[...] = s / jnp.sqrt(jnp.maximum(nsq, 1e-20))

    return pl.pallas_call(
        body,
        grid=(ACC_ROWS // BR,),
        in_specs=[pl.BlockSpec((NC, 3, BR), lambda i: (0, 0, i))],
        out_specs=pl.BlockSpec((3, BR), lambda i: (0, i)),
        out_shape=jax.ShapeDtypeStruct((3, ACC_ROWS), jnp.float32),
    )(partial)


def kernel(normals, weights, ray_indices, num_rays):
    idx = ray_indices.astype(jnp.int32)
    nx, ny, nz = (normals[:, 0], normals[:, 1], normals[:, 2])
    partial = _sc_segment_sum(nx, ny, nz, weights.reshape(N_SAMPLES), idx)
    merged = _merge_normalize(partial.reshape(NC, 3, ACC_ROWS))
    return merged[:, :N_RAYS].T
